# Initial kernel scaffold; baseline (speedup 1.0000x reference)
import jax
import jax.numpy as jnp
from jax.experimental import pallas as pl


def kernel(x, edge_index, action_x, W0, b0, W1, b1, W2, b2, Wm, bm, Wo, bo):
    raise NotImplementedError("write your pallas kernel here")


# speedup vs baseline: 6.0102x; 6.0102x over previous
import functools

import jax
import jax.numpy as jnp
from jax import lax
from jax.experimental import pallas as pl
from jax.experimental.pallas import tpu as pltpu
from jax.experimental.pallas import tpu_sc as plsc

N_PAD = 10496
E = 160000
E_PAD = 163840
D = 256
DH = 128
BM = 256
K_CH = 128
NSUB = 16
ROWS_W = N_PAD // NSUB
EP_SUB = E_PAD // NSUB
EP_W32 = E_PAD // 32
DEG_W = 128

_SC_MESH = dict(
    mesh=plsc.VectorSubcoreMesh(core_axis_name="c", subcore_axis_name="s",
                                num_cores=2, num_subcores=NSUB))


def _deg_body(col_hbm, ones_hbm, zeros_hbm, out_hbm, cidx, ones_v, acc):
    c = lax.axis_index("c")
    s = lax.axis_index("s")
    w = s * 2 + c
    pltpu.sync_copy(zeros_hbm.at[pl.ds(s * ROWS_W, ROWS_W)],
                    acc.at[pl.ds(s * ROWS_W, ROWS_W)])
    pltpu.sync_copy(ones_hbm, ones_v)
    plsc.subcore_barrier()

    def body(j, carry):
        off = w * EP_W32 + j * K_CH
        pltpu.sync_copy(col_hbm.at[pl.ds(off, K_CH)], cidx)
        pltpu.sync_copy(ones_v, acc.at[cidx], add=True)
        return carry

    lax.fori_loop(0, EP_W32 // K_CH, body, 0)
    plsc.subcore_barrier()
    pltpu.sync_copy(acc.at[pl.ds(s * ROWS_W, ROWS_W)],
                    out_hbm.at[c].at[pl.ds(s * ROWS_W, ROWS_W)])


def _agg_body(g_hbm, row_hbm, col_hbm, zeros_hbm, out_hbm,
              ridx, cidx, rows_v, sem, acc):
    c = lax.axis_index("c")
    s = lax.axis_index("s")
    pltpu.sync_copy(zeros_hbm.at[pl.ds(s * ROWS_W, ROWS_W)],
                    acc.at[pl.ds(s * ROWS_W, ROWS_W)])
    plsc.subcore_barrier()

    def body(j, carry):
        off = s * EP_SUB + j * K_CH
        pltpu.sync_copy(row_hbm.at[pl.ds(off, K_CH)], ridx)
        pltpu.sync_copy(col_hbm.at[pl.ds(off, K_CH)], cidx)
        pltpu.async_copy(g_hbm.at[c].at[ridx], rows_v, sem).wait()
        pltpu.sync_copy(rows_v, acc.at[cidx], add=True)
        return carry

    lax.fori_loop(0, EP_SUB // K_CH, body, 0)
    plsc.subcore_barrier()
    pltpu.sync_copy(acc.at[pl.ds(s * ROWS_W, ROWS_W)],
                    out_hbm.at[c].at[pl.ds(s * ROWS_W, ROWS_W)])


def _sc_deg(col_p, ones, zeros_deg):
    return pl.kernel(
        _deg_body,
        out_type=jax.ShapeDtypeStruct((2, N_PAD, DEG_W), jnp.float32),
        scratch_types=[
            pltpu.VMEM((K_CH,), jnp.int32),
            pltpu.VMEM((K_CH, DEG_W), jnp.float32),
            pltpu.VMEM_SHARED((N_PAD, DEG_W), jnp.float32),
        ],
        **_SC_MESH,
    )(col_p, ones, zeros_deg)


def _sc_agg(g, row_p, col_p, zeros_g):
    return pl.kernel(
        _agg_body,
        out_type=jax.ShapeDtypeStruct((2, N_PAD, DH), jnp.float32),
        scratch_types=[
            pltpu.VMEM((K_CH,), jnp.int32),
            pltpu.VMEM((K_CH,), jnp.int32),
            pltpu.VMEM((K_CH, DH), jnp.float32),
            pltpu.SemaphoreType.DMA,
            pltpu.VMEM_SHARED((N_PAD, DH), jnp.float32),
        ],
        **_SC_MESH,
    )(g, row_p, col_p, zeros_g)


def _lrelu(v):
    return jnp.where(v >= 0, v, 0.01 * v)


def _mm0_kbody(h_ref, w_ref, deg_ref, out_ref):
    dinv = lax.rsqrt(deg_ref[...])
    g = jnp.dot(h_ref[...], w_ref[...],
                preferred_element_type=jnp.float32) * dinv
    out_ref[0] = g[:, :DH]
    out_ref[1] = g[:, DH:]


def _mid_kbody(agg_ref, g_ref, deg_ref, b_ref, w_ref, out_ref):
    dinv = lax.rsqrt(deg_ref[...])
    aggc = jnp.concatenate([agg_ref[0], agg_ref[1]], axis=1)
    gc = jnp.concatenate([g_ref[0], g_ref[1]], axis=1)
    h = _lrelu(dinv * (aggc + gc) + b_ref[...])
    g = jnp.dot(h, w_ref[...], preferred_element_type=jnp.float32) * dinv
    out_ref[0] = g[:, :DH]
    out_ref[1] = g[:, DH:]


def _head_kbody(n_real, agg_ref, g_ref, deg_ref, b_ref, wm_ref, bm_ref,
                wo_ref, bo_ref, out_ref):
    dinv = lax.rsqrt(deg_ref[...])
    aggc = jnp.concatenate([agg_ref[0], agg_ref[1]], axis=1)
    gc = jnp.concatenate([g_ref[0], g_ref[1]], axis=1)
    h = _lrelu(dinv * (aggc + gc) + b_ref[...])
    hm = _lrelu(jnp.dot(h, wm_ref[...],
                        preferred_element_type=jnp.float32) + bm_ref[...])
    logits = jnp.dot(hm, wo_ref[...],
                     preferred_element_type=jnp.float32) + bo_ref[...]
    r = pl.program_id(0) * BM + lax.broadcasted_iota(jnp.int32, (BM, DH), 0)
    lane = lax.broadcasted_iota(jnp.int32, (BM, DH), 1)
    sel = (lane == 0) | (lane == 2)
    out_ref[...] = jnp.where(sel & (r >= n_real), -100.0, logits)


_GRID = N_PAD // BM


def _tc_mm0(h0, W0, deg_col):
    return pl.pallas_call(
        _mm0_kbody,
        grid=(_GRID,),
        in_specs=[
            pl.BlockSpec((BM, D), lambda i: (i, 0)),
            pl.BlockSpec((D, D), lambda i: (0, 0)),
            pl.BlockSpec((BM, 1), lambda i: (i, 0)),
        ],
        out_specs=pl.BlockSpec((2, BM, DH), lambda i: (0, i, 0)),
        out_shape=jax.ShapeDtypeStruct((2, N_PAD, DH), jnp.float32),
    )(h0, W0, deg_col)


def _tc_mid(agg, g, deg_col, b, W):
    return pl.pallas_call(
        _mid_kbody,
        grid=(_GRID,),
        in_specs=[
            pl.BlockSpec((2, BM, DH), lambda i: (0, i, 0)),
            pl.BlockSpec((2, BM, DH), lambda i: (0, i, 0)),
            pl.BlockSpec((BM, 1), lambda i: (i, 0)),
            pl.BlockSpec((1, D), lambda i: (0, 0)),
            pl.BlockSpec((D, D), lambda i: (0, 0)),
        ],
        out_specs=pl.BlockSpec((2, BM, DH), lambda i: (0, i, 0)),
        out_shape=jax.ShapeDtypeStruct((2, N_PAD, DH), jnp.float32),
    )(agg, g, deg_col, b, W)


def _tc_head(n_real, agg, g, deg_col, b2, Wm, bm, Wo_p, bo_p):
    return pl.pallas_call(
        functools.partial(_head_kbody, n_real),
        grid=(_GRID,),
        in_specs=[
            pl.BlockSpec((2, BM, DH), lambda i: (0, i, 0)),
            pl.BlockSpec((2, BM, DH), lambda i: (0, i, 0)),
            pl.BlockSpec((BM, 1), lambda i: (i, 0)),
            pl.BlockSpec((1, D), lambda i: (0, 0)),
            pl.BlockSpec((D, DH), lambda i: (0, 0)),
            pl.BlockSpec((1, DH), lambda i: (0, 0)),
            pl.BlockSpec((DH, DH), lambda i: (0, 0)),
            pl.BlockSpec((1, DH), lambda i: (0, 0)),
        ],
        out_specs=pl.BlockSpec((BM, DH), lambda i: (i, 0)),
        out_shape=jax.ShapeDtypeStruct((N_PAD, DH), jnp.float32),
    )(agg, g, deg_col, b2, Wm, bm, Wo_p, bo_p)


def kernel(x, edge_index, action_x, W0, b0, W1, b1, W2, b2, Wm, bm, Wo, bo):
    n_real = x.shape[0]
    n = n_real + action_x.shape[0]

    h0 = jnp.concatenate(
        [x, action_x, jnp.zeros((N_PAD - n, D), jnp.float32)], axis=0)
    row = edge_index[0].astype(jnp.int32)
    col = edge_index[1].astype(jnp.int32)
    pad_e = E_PAD - row.shape[0]
    row_p = jnp.concatenate([row, jnp.zeros((pad_e,), jnp.int32)])
    col_p = jnp.concatenate([col, jnp.full((pad_e,), n, jnp.int32)])

    ones = jnp.ones((K_CH, DEG_W), jnp.float32)
    zeros_g = jnp.zeros((N_PAD, DH), jnp.float32)

    deg2 = _sc_deg(col_p, ones, zeros_g)
    deg_col = (deg2[0, :, 0] + deg2[1, :, 0] + 1.0)[:, None]

    g0 = _tc_mm0(h0, W0, deg_col)
    agg0 = _sc_agg(g0, row_p, col_p, zeros_g)
    g1 = _tc_mid(agg0, g0, deg_col, b0[None, :], W1)
    agg1 = _sc_agg(g1, row_p, col_p, zeros_g)
    g2 = _tc_mid(agg1, g1, deg_col, b1[None, :], W2)
    agg2 = _sc_agg(g2, row_p, col_p, zeros_g)

    Wo_p = jnp.zeros((DH, DH), jnp.float32).at[:, :4].set(Wo)
    bo_p = jnp.zeros((1, DH), jnp.float32).at[0, :4].set(bo)
    out = _tc_head(n_real, agg2, g2, deg_col, b2[None, :], Wm, bm[None, :],
                   Wo_p, bo_p)

    return ((out[:n, 0], out[:n, 1]), (out[:n, 2], out[:n, 3]))


# speedup vs baseline: 8.0825x; 1.3448x over previous
import functools

import jax
import jax.numpy as jnp
from jax import lax
from jax.experimental import pallas as pl
from jax.experimental.pallas import tpu as pltpu
from jax.experimental.pallas import tpu_sc as plsc

N_PAD = 10496
E = 160000
E_PAD = 163840
D = 256
DH = 128
BM = 256
K_CH = 128
NSUB = 16
ROWS_W = N_PAD // NSUB
EP_SUB = E_PAD // NSUB
NCH = EP_SUB // K_CH
NCH2 = NCH // 2
EP_W32 = E_PAD // 32
DEG_W = 128

_SC_MESH = dict(
    mesh=plsc.VectorSubcoreMesh(core_axis_name="c", subcore_axis_name="s",
                                num_cores=2, num_subcores=NSUB))


def _deg_body(col_hbm, ones_hbm, zeros_hbm, out_hbm, cidx, ones_v, acc):
    c = lax.axis_index("c")
    s = lax.axis_index("s")
    w = s * 2 + c
    pltpu.sync_copy(zeros_hbm.at[pl.ds(s * ROWS_W, ROWS_W)],
                    acc.at[pl.ds(s * ROWS_W, ROWS_W)])
    pltpu.sync_copy(ones_hbm, ones_v)
    plsc.subcore_barrier()

    def body(j, carry):
        off = w * EP_W32 + j * K_CH
        pltpu.sync_copy(col_hbm.at[pl.ds(off, K_CH)], cidx)
        pltpu.sync_copy(ones_v, acc.at[cidx], add=True)
        return carry

    lax.fori_loop(0, EP_W32 // K_CH, body, 0)
    plsc.subcore_barrier()
    pltpu.sync_copy(acc.at[pl.ds(s * ROWS_W, ROWS_W)],
                    out_hbm.at[c].at[pl.ds(s * ROWS_W, ROWS_W)])


def _agg_body(g_hbm, row3_hbm, col3_hbm, zeros_hbm, out_hbm,
              idx_r, idx_c, buf0, buf1, sem0, sem1, acc):
    c = lax.axis_index("c")
    s = lax.axis_index("s")
    pltpu.sync_copy(zeros_hbm.at[pl.ds(s * ROWS_W, ROWS_W)],
                    acc.at[pl.ds(s * ROWS_W, ROWS_W)])
    plsc.subcore_barrier()

    gsrc = g_hbm.at[c]

    def phase(p, carry):
        pltpu.sync_copy(row3_hbm.at[s, pl.ds(p * NCH2, NCH2)], idx_r)
        pltpu.sync_copy(col3_hbm.at[s, pl.ds(p * NCH2, NCH2)], idx_c)
        pltpu.async_copy(gsrc.at[idx_r.at[0]], buf0, sem0)
        pltpu.async_copy(gsrc.at[idx_r.at[1]], buf1, sem1)

        def body(j2, carry2):
            j = j2 * 2
            for b, buf, sem in ((0, buf0, sem0), (1, buf1, sem1)):
                pltpu.make_async_copy(
                    gsrc.at[idx_r.at[j + b]], buf, sem).wait()
                pltpu.sync_copy(buf, acc.at[idx_c.at[j + b]], add=True)

                @pl.when(j + b + 2 < NCH2)
                def _():
                    pltpu.async_copy(gsrc.at[idx_r.at[j + b + 2]], buf, sem)
            return carry2

        lax.fori_loop(0, NCH2 // 2, body, 0)
        return carry

    lax.fori_loop(0, 2, phase, 0)
    plsc.subcore_barrier()
    pltpu.sync_copy(acc.at[pl.ds(s * ROWS_W, ROWS_W)],
                    out_hbm.at[c].at[pl.ds(s * ROWS_W, ROWS_W)])


def _sc_deg(col_p, ones, zeros_deg):
    return pl.kernel(
        _deg_body,
        out_type=jax.ShapeDtypeStruct((2, N_PAD, DEG_W), jnp.float32),
        scratch_types=[
            pltpu.VMEM((K_CH,), jnp.int32),
            pltpu.VMEM((K_CH, DEG_W), jnp.float32),
            pltpu.VMEM_SHARED((N_PAD, DEG_W), jnp.float32),
        ],
        **_SC_MESH,
    )(col_p, ones, zeros_deg)


def _sc_agg(g, row3, col3, zeros_g):
    return pl.kernel(
        _agg_body,
        out_type=jax.ShapeDtypeStruct((2, N_PAD, DH), jnp.float32),
        scratch_types=[
            pltpu.VMEM((NCH2, K_CH), jnp.int32),
            pltpu.VMEM((NCH2, K_CH), jnp.int32),
            pltpu.VMEM((K_CH, DH), jnp.float32),
            pltpu.VMEM((K_CH, DH), jnp.float32),
            pltpu.SemaphoreType.DMA,
            pltpu.SemaphoreType.DMA,
            pltpu.VMEM_SHARED((N_PAD, DH), jnp.float32),
        ],
        **_SC_MESH,
    )(g, row3, col3, zeros_g)


def _lrelu(v):
    return jnp.where(v >= 0, v, 0.01 * v)


def _mm0_kbody(h_ref, w_ref, deg_ref, out_ref):
    dinv = lax.rsqrt(deg_ref[...])
    g = jnp.dot(h_ref[...], w_ref[...],
                preferred_element_type=jnp.float32) * dinv
    out_ref[0] = g[:, :DH]
    out_ref[1] = g[:, DH:]


def _mid_kbody(agg_ref, g_ref, deg_ref, b_ref, w_ref, out_ref):
    dinv = lax.rsqrt(deg_ref[...])
    aggc = jnp.concatenate([agg_ref[0], agg_ref[1]], axis=1)
    gc = jnp.concatenate([g_ref[0], g_ref[1]], axis=1)
    h = _lrelu(dinv * (aggc + gc) + b_ref[...])
    g = jnp.dot(h, w_ref[...], preferred_element_type=jnp.float32) * dinv
    out_ref[0] = g[:, :DH]
    out_ref[1] = g[:, DH:]


def _head_kbody(n_real, agg_ref, g_ref, deg_ref, b_ref, wm_ref, bm_ref,
                wo_ref, bo_ref, out_ref):
    dinv = lax.rsqrt(deg_ref[...])
    aggc = jnp.concatenate([agg_ref[0], agg_ref[1]], axis=1)
    gc = jnp.concatenate([g_ref[0], g_ref[1]], axis=1)
    h = _lrelu(dinv * (aggc + gc) + b_ref[...])
    hm = _lrelu(jnp.dot(h, wm_ref[...],
                        preferred_element_type=jnp.float32) + bm_ref[...])
    logits = jnp.dot(hm, wo_ref[...],
                     preferred_element_type=jnp.float32) + bo_ref[...]
    r = pl.program_id(0) * BM + lax.broadcasted_iota(jnp.int32, (BM, DH), 0)
    lane = lax.broadcasted_iota(jnp.int32, (BM, DH), 1)
    sel = (lane == 0) | (lane == 2)
    out_ref[...] = jnp.where(sel & (r >= n_real), -100.0, logits)


_GRID = N_PAD // BM


def _tc_mm0(h0, W0, deg_col):
    return pl.pallas_call(
        _mm0_kbody,
        grid=(_GRID,),
        in_specs=[
            pl.BlockSpec((BM, D), lambda i: (i, 0)),
            pl.BlockSpec((D, D), lambda i: (0, 0)),
            pl.BlockSpec((BM, 1), lambda i: (i, 0)),
        ],
        out_specs=pl.BlockSpec((2, BM, DH), lambda i: (0, i, 0)),
        out_shape=jax.ShapeDtypeStruct((2, N_PAD, DH), jnp.float32),
    )(h0, W0, deg_col)


def _tc_mid(agg, g, deg_col, b, W):
    return pl.pallas_call(
        _mid_kbody,
        grid=(_GRID,),
        in_specs=[
            pl.BlockSpec((2, BM, DH), lambda i: (0, i, 0)),
            pl.BlockSpec((2, BM, DH), lambda i: (0, i, 0)),
            pl.BlockSpec((BM, 1), lambda i: (i, 0)),
            pl.BlockSpec((1, D), lambda i: (0, 0)),
            pl.BlockSpec((D, D), lambda i: (0, 0)),
        ],
        out_specs=pl.BlockSpec((2, BM, DH), lambda i: (0, i, 0)),
        out_shape=jax.ShapeDtypeStruct((2, N_PAD, DH), jnp.float32),
    )(agg, g, deg_col, b, W)


def _tc_head(n_real, agg, g, deg_col, b2, Wm, bm, Wo_p, bo_p):
    return pl.pallas_call(
        functools.partial(_head_kbody, n_real),
        grid=(_GRID,),
        in_specs=[
            pl.BlockSpec((2, BM, DH), lambda i: (0, i, 0)),
            pl.BlockSpec((2, BM, DH), lambda i: (0, i, 0)),
            pl.BlockSpec((BM, 1), lambda i: (i, 0)),
            pl.BlockSpec((1, D), lambda i: (0, 0)),
            pl.BlockSpec((D, DH), lambda i: (0, 0)),
            pl.BlockSpec((1, DH), lambda i: (0, 0)),
            pl.BlockSpec((DH, DH), lambda i: (0, 0)),
            pl.BlockSpec((1, DH), lambda i: (0, 0)),
        ],
        out_specs=pl.BlockSpec((BM, DH), lambda i: (i, 0)),
        out_shape=jax.ShapeDtypeStruct((N_PAD, DH), jnp.float32),
    )(agg, g, deg_col, b2, Wm, bm, Wo_p, bo_p)


def kernel(x, edge_index, action_x, W0, b0, W1, b1, W2, b2, Wm, bm, Wo, bo):
    n_real = x.shape[0]
    n = n_real + action_x.shape[0]

    h0 = jnp.concatenate(
        [x, action_x, jnp.zeros((N_PAD - n, D), jnp.float32)], axis=0)
    row = edge_index[0].astype(jnp.int32)
    col = edge_index[1].astype(jnp.int32)
    pad_e = E_PAD - row.shape[0]
    row_p = jnp.concatenate([row, jnp.zeros((pad_e,), jnp.int32)])
    col_p = jnp.concatenate([col, jnp.full((pad_e,), n, jnp.int32)])
    row3 = row_p.reshape(NSUB, NCH, K_CH)
    col3 = col_p.reshape(NSUB, NCH, K_CH)

    ones = jnp.ones((K_CH, DEG_W), jnp.float32)
    zeros_g = jnp.zeros((N_PAD, DH), jnp.float32)

    deg2 = _sc_deg(col_p, ones, zeros_g)
    deg_col = (deg2[0, :, 0] + deg2[1, :, 0] + 1.0)[:, None]

    g0 = _tc_mm0(h0, W0, deg_col)
    agg0 = _sc_agg(g0, row3, col3, zeros_g)
    g1 = _tc_mid(agg0, g0, deg_col, b0[None, :], W1)
    agg1 = _sc_agg(g1, row3, col3, zeros_g)
    g2 = _tc_mid(agg1, g1, deg_col, b1[None, :], W2)
    agg2 = _sc_agg(g2, row3, col3, zeros_g)

    Wo_p = jnp.zeros((DH, DH), jnp.float32).at[:, :4].set(Wo)
    bo_p = jnp.zeros((1, DH), jnp.float32).at[0, :4].set(bo)
    out = _tc_head(n_real, agg2, g2, deg_col, b2[None, :], Wm, bm[None, :],
                   Wo_p, bo_p)

    return ((out[:n, 0], out[:n, 1]), (out[:n, 2], out[:n, 3]))
